# baseline (speedup 1.0000x reference)
import functools

import jax
import jax.numpy as jnp
from jax import lax
from jax.experimental import pallas as pl
from jax.experimental.pallas import tpu as pltpu
from jax.experimental.pallas import tpu_sc as plsc

D_MODEL = 2048
NUM_CORES = 2
NUM_SUBCORES = 16
NUM_WORKERS = NUM_CORES * NUM_SUBCORES
CHUNK = 16
NBUF = 3


def _make_lookup(n_idx: int, d: int):
  assert n_idx % (8 * NUM_WORKERS) == 0
  per_w = n_idx // NUM_WORKERS
  assert per_w % CHUNK == 0 and CHUNK % 8 == 0
  n_chunks = per_w // CHUNK
  assert n_chunks >= 2 * NBUF
  main = (n_chunks // NBUF) * NBUF
  assert main >= n_chunks - NBUF + 1

  mesh = plsc.VectorSubcoreMesh(
      core_axis_name="c", subcore_axis_name="s",
      num_cores=NUM_CORES, num_subcores=NUM_SUBCORES)

  @functools.partial(
      pl.kernel,
      out_type=jax.ShapeDtypeStruct((n_idx, d), jnp.float32),
      mesh=mesh,
      scratch_types=[
          pltpu.VMEM((per_w,), jnp.int32),
          [pltpu.VMEM((CHUNK, d), jnp.float32) for _ in range(NBUF)],
          [pltpu.SemaphoreType.DMA for _ in range(NBUF)],
          [pltpu.SemaphoreType.DMA for _ in range(NBUF)],
      ],
  )
  def lookup(table_hbm, idx_hbm, out_hbm, idx_v, bufs, gsems, wsems):
    wid = lax.axis_index("s") * NUM_CORES + lax.axis_index("c")
    base = wid * per_w
    pltpu.sync_copy(idx_hbm.at[pl.ds(base, per_w)], idx_v)

    def gather(jj, b):
      return pltpu.make_async_copy(
          table_hbm.at[idx_v.at[pl.ds(jj * CHUNK, CHUNK)]], bufs[b], gsems[b])

    def writeback(jj, b):
      return pltpu.make_async_copy(
          bufs[b], out_hbm.at[pl.ds(base + jj * CHUNK, CHUNK)], wsems[b])

    for b in range(NBUF - 1):
      gather(b, b).start()

    @pl.loop(0, main, step=NBUF)
    def _(j):
      for b in range(NBUF):
        jj = j + b
        gather(jj, b).wait()
        writeback(jj, b).start()
        fut = jj + NBUF - 1
        fb = (b + NBUF - 1) % NBUF

        @pl.when((fut < n_chunks) & (jj >= 1))
        def _():
          writeback(jj - 1, fb).wait()

        @pl.when(fut < n_chunks)
        def _():
          gather(fut, fb).start()

    for jj in range(main, n_chunks):
      gather(jj, jj % NBUF).wait()
      writeback(jj, jj % NBUF).start()

    for i in range(NBUF):
      jj = n_chunks - NBUF + i
      writeback(jj, jj % NBUF).wait()

  return lookup


def kernel(ipt, table):
  b, s = ipt.shape
  v, d = table.shape
  idx = ipt.reshape(b * s).astype(jnp.int32)
  out = _make_lookup(b * s, d)(table, idx)
  return out.reshape(b, s, d)


# speedup vs baseline: 1.0084x; 1.0084x over previous
import functools

import jax
import jax.numpy as jnp
from jax import lax
from jax.experimental import pallas as pl
from jax.experimental.pallas import tpu as pltpu
from jax.experimental.pallas import tpu_sc as plsc

D_MODEL = 2048
NUM_CORES = 2
NUM_SUBCORES = 16
NUM_WORKERS = NUM_CORES * NUM_SUBCORES
CHUNK = 8
NBUF = 4


def _make_lookup(n_idx: int, d: int):
  assert n_idx % (8 * NUM_WORKERS) == 0
  per_w = n_idx // NUM_WORKERS
  assert per_w % (NBUF * CHUNK) == 0
  n_chunks = per_w // CHUNK
  assert n_chunks >= 2 * NBUF

  mesh = plsc.VectorSubcoreMesh(
      core_axis_name="c", subcore_axis_name="s",
      num_cores=NUM_CORES, num_subcores=NUM_SUBCORES)

  @functools.partial(
      pl.kernel,
      out_type=jax.ShapeDtypeStruct((n_idx, d), jnp.float32),
      mesh=mesh,
      scratch_types=[
          pltpu.VMEM((per_w,), jnp.int32),
          [pltpu.VMEM((CHUNK, d), jnp.float32) for _ in range(NBUF)],
          [pltpu.SemaphoreType.DMA for _ in range(NBUF)],
          [pltpu.SemaphoreType.DMA for _ in range(NBUF)],
      ],
  )
  def lookup(table_hbm, idx_hbm, out_hbm, idx_v, bufs, gsems, wsems):
    wid = lax.axis_index("s") * NUM_CORES + lax.axis_index("c")
    base = wid * per_w
    pltpu.sync_copy(idx_hbm.at[pl.ds(base, per_w)], idx_v)

    def gather(jj, b):
      return pltpu.make_async_copy(
          table_hbm.at[idx_v.at[pl.ds(jj * CHUNK, CHUNK)]], bufs[b], gsems[b])

    def writeback(jj, b):
      return pltpu.make_async_copy(
          bufs[b], out_hbm.at[pl.ds(base + jj * CHUNK, CHUNK)], wsems[b])

    for b in range(NBUF - 1):
      gather(b, b).start()

    @pl.loop(0, n_chunks, step=NBUF)
    def _(j):
      for b in range(NBUF):
        jj = j + b
        gather(jj, b).wait()
        writeback(jj, b).start()
        fut = jj + NBUF - 1
        fb = (b + NBUF - 1) % NBUF
        pb = (b + NBUF - 1) % NBUF

        @pl.when((fut < n_chunks) & (jj >= 1))
        def _():
          writeback(jj - 1, pb).wait()

        @pl.when(fut < n_chunks)
        def _():
          gather(fut, fb).start()

    for i in range(NBUF):
      jj = n_chunks - NBUF + i
      writeback(jj, jj % NBUF).wait()

  return lookup


def kernel(ipt, table):
  b, s = ipt.shape
  v, d = table.shape
  idx = ipt.reshape(b * s).astype(jnp.int32)
  out = _make_lookup(b * s, d)(table, idx)
  return out.reshape(b, s, d)


# speedup vs baseline: 1.0153x; 1.0069x over previous
import functools

import jax
import jax.numpy as jnp
from jax import lax
from jax.experimental import pallas as pl
from jax.experimental.pallas import tpu as pltpu
from jax.experimental.pallas import tpu_sc as plsc

D_MODEL = 2048
NUM_CORES = 2
NUM_SUBCORES = 16
NUM_WORKERS = NUM_CORES * NUM_SUBCORES
CHUNK = 8
NBUF = 4


def _make_lookup(n_idx: int, d: int):
  assert n_idx % (8 * NUM_WORKERS) == 0
  per_w = n_idx // NUM_WORKERS
  assert per_w % (NBUF * CHUNK) == 0
  n_chunks = per_w // CHUNK
  assert n_chunks >= 2 * NBUF

  mesh = plsc.VectorSubcoreMesh(
      core_axis_name="c", subcore_axis_name="s",
      num_cores=NUM_CORES, num_subcores=NUM_SUBCORES)

  @functools.partial(
      pl.kernel,
      out_type=jax.ShapeDtypeStruct((n_idx, d), jnp.float32),
      mesh=mesh,
      scratch_types=[
          pltpu.VMEM((per_w,), jnp.int32),
          [pltpu.VMEM((CHUNK, d), jnp.float32) for _ in range(NBUF)],
          [pltpu.SemaphoreType.DMA for _ in range(NBUF)],
          [pltpu.SemaphoreType.DMA for _ in range(NBUF)],
      ],
  )
  def lookup(table_hbm, idx_hbm, out_hbm, idx_v, bufs, gsems, wsems):
    wid = lax.axis_index("s") * NUM_CORES + lax.axis_index("c")
    base = wid * per_w
    pltpu.sync_copy(idx_hbm.at[pl.ds(base, per_w)], idx_v)

    def gather(jj, b):
      return pltpu.make_async_copy(
          table_hbm.at[idx_v.at[pl.ds(jj * CHUNK, CHUNK)]], bufs[b], gsems[b])

    def writeback(jj, b):
      return pltpu.make_async_copy(
          bufs[b], out_hbm.at[pl.ds(base + jj * CHUNK, CHUNK)], wsems[b])

    for b in range(NBUF - 2):
      gather(b, b).start()

    @pl.loop(0, n_chunks, step=NBUF)
    def _(j):
      for b in range(NBUF):
        jj = j + b
        gather(jj, b).wait()
        writeback(jj, b).start()
        fut = jj + NBUF - 2
        fb = (b + NBUF - 2) % NBUF

        @pl.when((fut < n_chunks) & (jj >= 2))
        def _():
          writeback(jj - 2, fb).wait()

        @pl.when(fut < n_chunks)
        def _():
          gather(fut, fb).start()

    for i in range(NBUF):
      jj = n_chunks - NBUF + i
      writeback(jj, jj % NBUF).wait()

  return lookup


def kernel(ipt, table):
  b, s = ipt.shape
  v, d = table.shape
  idx = ipt.reshape(b * s).astype(jnp.int32)
  out = _make_lookup(b * s, d)(table, idx)
  return out.reshape(b, s, d)
